# baseline (speedup 1.0000x reference)
import functools

import jax
import jax.numpy as jnp
from jax import lax
from jax.experimental import pallas as pl
from jax.experimental.pallas import tpu as pltpu

_BN_EPS = 1e-5
_OUT_SIZE = 601


def _body(x_ref, ws_ref, wr_ref, g_ref, b_ref, w0_ref, w1_ref, w2_ref,
          fb_ref, state_ref, reward_ref, *, nb, n, h, w, c, cp):
    rows = n * h * w
    bf = jnp.bfloat16
    f32 = jnp.float32

    rr = lax.broadcasted_iota(jnp.int32, (rows, 1), 0)
    hh = (rr // w) % h
    ww = rr % w
    m_hm = (hh >= 1).astype(bf)
    m_hp = (hh <= h - 2).astype(bf)
    m_wl = (ww >= 1).astype(f32)
    m_wr = (ww <= w - 2).astype(f32)

    def catw(tap9):
        return [jnp.concatenate([tap9[dx], tap9[3 + dx], tap9[6 + dx]], axis=0)
                for dx in range(3)]

    def conv(a, wcat):
        ab = a.astype(bf)
        up = jnp.roll(ab, w, axis=0) * m_hm
        dn = jnp.roll(ab, -w, axis=0) * m_hp
        x3 = jnp.concatenate([up, ab, dn], axis=1)

        def mm(dx):
            return jnp.dot(x3, wcat[dx], preferred_element_type=f32)

        return (mm(1) + jnp.roll(mm(0), 1, axis=0) * m_wl
                + jnp.roll(mm(2), -1, axis=0) * m_wr)

    def bn(y, i):
        mu = jnp.mean(y, axis=0, keepdims=True)
        var = jnp.mean((y - mu) ** 2, axis=0, keepdims=True)
        return (y - mu) * lax.rsqrt(var + _BN_EPS) * g_ref[i:i + 1, :] \
            + b_ref[i:i + 1, :]

    wsb = ws_ref[...].astype(bf)
    out = jnp.maximum(bn(conv(x_ref[...], catw(wsb)), 0), 0.0)
    wrb = wr_ref[...].astype(bf)
    for blk in range(nb):
        resid = out
        y = conv(out, catw(wrb[(2 * blk) * 9:(2 * blk) * 9 + 9]))
        y = jnp.maximum(bn(y, 1 + 2 * blk), 0.0)
        y = conv(y, catw(wrb[(2 * blk + 1) * 9:(2 * blk + 1) * 9 + 9]))
        y = bn(y, 2 + 2 * blk)
        out = jnp.maximum(y + resid, 0.0)
    state_ref[...] = out.T

    ho, wo = h // 2, w // 2
    prows = n * ho * wo
    rp = lax.broadcasted_iota(jnp.int32, (prows, rows), 0)
    q = lax.broadcasted_iota(jnp.int32, (prows, rows), 1)
    n_r = rp % n
    s_r = rp // n
    n_q = q // (h * w)
    rem_q = q % (h * w)
    pmat = (jnp.where(n_q == n_r, 0.25, 0.0)
            * jnp.where((rem_q // w) // 2 == s_r // wo, 1.0, 0.0)
            * jnp.where((rem_q % w) // 2 == s_r % wo, 1.0, 0.0)
            ).astype(bf)
    pooled = jnp.dot(pmat, out.astype(bf), preferred_element_type=f32)

    w0b = w0_ref[...].astype(bf)
    d1 = w0b.shape[1]
    h1 = jnp.zeros((n, d1), f32)
    for s in range(prows // n):
        h1 = h1 + jnp.dot(pooled[s * n:(s + 1) * n, :].astype(bf),
                          w0b[s * c:(s + 1) * c, :],
                          preferred_element_type=f32)
    h1 = jnp.maximum(h1 + fb_ref[0:1, :d1], 0.0)
    h2 = jnp.dot(h1.astype(bf), w1_ref[...].astype(bf),
                 preferred_element_type=f32)
    h2 = jnp.maximum(h2 + fb_ref[1:2, :w1_ref.shape[1]], 0.0)
    o = jnp.dot(h2.astype(bf), w2_ref[...].astype(bf),
                preferred_element_type=f32)
    reward_ref[...] = o + fb_ref[2:3, :]


def kernel(x, stem_w, rb_w, bn_gamma, bn_beta, fc_w0, fc_w1, fc_w2, fc_b):
    n, c_in, h, w = x.shape
    c = stem_w.shape[2]
    nb = rb_w.shape[0] // 18
    rows = n * h * w
    prows = n * (h // 2) * (w // 2)
    npos = prows // n
    cp = ((c_in + 127) // 128) * 128

    x2d = x.transpose(0, 2, 3, 1).reshape(rows, c_in)
    x2d = jnp.pad(x2d, ((0, 0), (0, cp - c_in)))
    ws = jnp.pad(stem_w, ((0, 0), (0, cp - c_in), (0, 0)))
    w0p = fc_w0.reshape(c, npos, fc_w0.shape[1]).transpose(1, 0, 2)
    w0p = w0p.reshape(npos * c, fc_w0.shape[1])

    out_pad = fc_w2.shape[1]
    state2d, reward_p = pl.pallas_call(
        functools.partial(_body, nb=nb, n=n, h=h, w=w, c=c, cp=cp),
        out_shape=(jax.ShapeDtypeStruct((c, rows), jnp.float32),
                   jax.ShapeDtypeStruct((n, out_pad), jnp.float32)),
        grid=(1,),
        in_specs=[
            pl.BlockSpec((rows, cp), lambda i: (0, 0)),
            pl.BlockSpec((9, cp, c), lambda i: (0, 0, 0)),
            pl.BlockSpec(rb_w.shape, lambda i: (0, 0, 0)),
            pl.BlockSpec(bn_gamma.shape, lambda i: (0, 0)),
            pl.BlockSpec(bn_beta.shape, lambda i: (0, 0)),
            pl.BlockSpec(w0p.shape, lambda i: (0, 0)),
            pl.BlockSpec(fc_w1.shape, lambda i: (0, 0)),
            pl.BlockSpec(fc_w2.shape, lambda i: (0, 0)),
            pl.BlockSpec(fc_b.shape, lambda i: (0, 0)),
        ],
        out_specs=(pl.BlockSpec((c, rows), lambda i: (0, 0)),
                   pl.BlockSpec((n, out_pad), lambda i: (0, 0))),
        compiler_params=pltpu.CompilerParams(
            dimension_semantics=("arbitrary",)),
    )(x2d, ws, rb_w, bn_gamma, bn_beta, w0p, fc_w1, fc_w2, fc_b)

    state = state2d.reshape(c, n, h, w).transpose(1, 0, 2, 3)
    return state, reward_p[:, :_OUT_SIZE]


# speedup vs baseline: 1.1928x; 1.1928x over previous
import functools

import jax
import jax.numpy as jnp
import numpy as np
from jax import lax
from jax.experimental import pallas as pl
from jax.experimental.pallas import tpu as pltpu

_BN_EPS = 1e-5
_OUT_SIZE = 601


def _body(x_ref, ws_ref, wr_ref, g_ref, b_ref, pm_ref, w0_ref, w1_ref,
          w2_ref, fb_ref, state_ref, reward_ref, *, nb, n, h, w, c, cp):
    rows = n * h * w
    bf = jnp.bfloat16
    f32 = jnp.float32

    rr = lax.broadcasted_iota(jnp.int32, (rows, 1), 0)
    hh = (rr // w) % h
    ww = rr % w
    m_hm = (hh >= 1).astype(bf)
    m_hp = (hh <= h - 2).astype(bf)
    m_wl = (ww >= 1).astype(f32)
    m_wr = (ww <= w - 2).astype(f32)

    def catw(tap9):
        cols = [jnp.concatenate([tap9[dx], tap9[3 + dx], tap9[6 + dx]],
                                axis=0) for dx in range(3)]
        return jnp.concatenate(cols, axis=1)

    def conv(a, wcat):
        ab = a.astype(bf)
        up = jnp.roll(ab, w, axis=0) * m_hm
        dn = jnp.roll(ab, -w, axis=0) * m_hp
        x3 = jnp.concatenate([up, ab, dn], axis=1)
        u = jnp.dot(x3, wcat, preferred_element_type=f32)
        return (u[:, c:2 * c] + jnp.roll(u[:, :c], 1, axis=0) * m_wl
                + jnp.roll(u[:, 2 * c:], -1, axis=0) * m_wr)

    def bn(y, i):
        mu = jnp.mean(y, axis=0, keepdims=True)
        var = jnp.mean(y * y, axis=0, keepdims=True) - mu * mu
        s = lax.rsqrt(var + _BN_EPS) * g_ref[i:i + 1, :]
        return y * s + (b_ref[i:i + 1, :] - mu * s)

    wsb = ws_ref[...].astype(bf)
    out = jnp.maximum(bn(conv(x_ref[...], catw(wsb)), 0), 0.0)
    wrb = wr_ref[...].astype(bf)
    for blk in range(nb):
        resid = out
        y = conv(out, catw(wrb[(2 * blk) * 9:(2 * blk) * 9 + 9]))
        y = jnp.maximum(bn(y, 1 + 2 * blk), 0.0)
        y = conv(y, catw(wrb[(2 * blk + 1) * 9:(2 * blk + 1) * 9 + 9]))
        y = bn(y, 2 + 2 * blk)
        out = jnp.maximum(y + resid, 0.0)
    state_ref[...] = out

    pooled = jnp.dot(pm_ref[...], out.astype(bf), preferred_element_type=f32)

    w0b = w0_ref[...].astype(bf)
    d1 = w0b.shape[1]
    npos = pm_ref.shape[0] // n
    h1 = jnp.zeros((n, d1), f32)
    for s in range(npos):
        h1 = h1 + jnp.dot(pooled[s * n:(s + 1) * n, :].astype(bf),
                          w0b[s * c:(s + 1) * c, :],
                          preferred_element_type=f32)
    h1 = jnp.maximum(h1 + fb_ref[0:1, :d1], 0.0)
    h2 = jnp.dot(h1.astype(bf), w1_ref[...].astype(bf),
                 preferred_element_type=f32)
    h2 = jnp.maximum(h2 + fb_ref[1:2, :w1_ref.shape[1]], 0.0)
    o = jnp.dot(h2.astype(bf), w2_ref[...].astype(bf),
                preferred_element_type=f32)
    reward_ref[...] = o + fb_ref[2:3, :]


def _pool_const(n, h, w):
    ho, wo = h // 2, w // 2
    rp = np.arange(n * ho * wo)[:, None]
    q = np.arange(n * h * w)[None, :]
    n_r, s_r = rp % n, rp // n
    n_q, rem_q = q // (h * w), q % (h * w)
    hit = ((n_q == n_r) & ((rem_q // w) // 2 == s_r // wo)
           & ((rem_q % w) // 2 == s_r % wo))
    return np.where(hit, 0.25, 0.0).astype(jnp.bfloat16)


def kernel(x, stem_w, rb_w, bn_gamma, bn_beta, fc_w0, fc_w1, fc_w2, fc_b):
    n, c_in, h, w = x.shape
    c = stem_w.shape[2]
    nb = rb_w.shape[0] // 18
    rows = n * h * w
    prows = n * (h // 2) * (w // 2)
    npos = prows // n
    cp = ((c_in + 127) // 128) * 128

    x2d = x.transpose(0, 2, 3, 1).reshape(rows, c_in)
    x2d = jnp.pad(x2d, ((0, 0), (0, cp - c_in)))
    ws = jnp.pad(stem_w, ((0, 0), (0, cp - c_in), (0, 0)))
    w0p = fc_w0.reshape(c, npos, fc_w0.shape[1]).transpose(1, 0, 2)
    w0p = w0p.reshape(npos * c, fc_w0.shape[1])
    pmat = _pool_const(n, h, w)

    out_pad = fc_w2.shape[1]
    state2d, reward_p = pl.pallas_call(
        functools.partial(_body, nb=nb, n=n, h=h, w=w, c=c, cp=cp),
        out_shape=(jax.ShapeDtypeStruct((rows, c), jnp.float32),
                   jax.ShapeDtypeStruct((n, out_pad), jnp.float32)),
        grid=(1,),
        in_specs=[
            pl.BlockSpec((rows, cp), lambda i: (0, 0)),
            pl.BlockSpec((9, cp, c), lambda i: (0, 0, 0)),
            pl.BlockSpec(rb_w.shape, lambda i: (0, 0, 0)),
            pl.BlockSpec(bn_gamma.shape, lambda i: (0, 0)),
            pl.BlockSpec(bn_beta.shape, lambda i: (0, 0)),
            pl.BlockSpec(pmat.shape, lambda i: (0, 0)),
            pl.BlockSpec(w0p.shape, lambda i: (0, 0)),
            pl.BlockSpec(fc_w1.shape, lambda i: (0, 0)),
            pl.BlockSpec(fc_w2.shape, lambda i: (0, 0)),
            pl.BlockSpec(fc_b.shape, lambda i: (0, 0)),
        ],
        out_specs=(pl.BlockSpec((rows, c), lambda i: (0, 0)),
                   pl.BlockSpec((n, out_pad), lambda i: (0, 0))),
        compiler_params=pltpu.CompilerParams(
            dimension_semantics=("arbitrary",)),
    )(x2d, ws, rb_w, bn_gamma, bn_beta, pmat, w0p, fc_w1, fc_w2, fc_b)

    state = state2d.reshape(n, h, w, c).transpose(0, 3, 1, 2)
    return state, reward_p[:, :_OUT_SIZE]


# speedup vs baseline: 1.2247x; 1.0267x over previous
import functools

import jax
import jax.numpy as jnp
import numpy as np
from jax import lax
from jax.experimental import pallas as pl
from jax.experimental.pallas import tpu as pltpu

_BN_EPS = 1e-5
_OUT_SIZE = 601


def _body(x_ref, ws_ref, wr_ref, g_ref, b_ref, pm_ref, w0_ref, w1_ref,
          w2_ref, fb_ref, state_ref, reward_ref, *, nb, n, h, w, c, cp):
    rows = n * h * w
    bf = jnp.bfloat16
    f32 = jnp.float32

    rr = lax.broadcasted_iota(jnp.int32, (rows, 1), 0)
    hh = (rr // w) % h
    ww = rr % w
    m_hm = (hh >= 1).astype(bf)
    m_hp = (hh <= h - 2).astype(bf)
    m_wl = (ww >= 1).astype(f32)
    m_wr = (ww <= w - 2).astype(f32)

    def catw(tap9):
        cols = [jnp.concatenate([tap9[dx], tap9[3 + dx], tap9[6 + dx]],
                                axis=0) for dx in range(3)]
        return jnp.concatenate(cols, axis=1)

    def conv(a, wcat):
        ab = a.astype(bf)
        up = jnp.roll(ab, w, axis=0) * m_hm
        dn = jnp.roll(ab, -w, axis=0) * m_hp
        x3 = jnp.concatenate([up, ab, dn], axis=1)
        u = jnp.dot(x3, wcat, preferred_element_type=f32)
        return (u[:, c:2 * c] + jnp.roll(u[:, :c], 1, axis=0) * m_wl
                + jnp.roll(u[:, 2 * c:], -1, axis=0) * m_wr)

    def bn(y, i):
        mu = jnp.mean(y, axis=0, keepdims=True)
        var = jnp.mean(y * y, axis=0, keepdims=True) - mu * mu
        s = lax.rsqrt(var + _BN_EPS) * g_ref[i:i + 1, :]
        return y * s + (b_ref[i:i + 1, :] - mu * s)

    wsb = ws_ref[...].astype(bf)
    out = jnp.maximum(bn(conv(x_ref[...], catw(wsb)), 0), 0.0)
    wrb = wr_ref[...].astype(bf)
    for blk in range(nb):
        resid = out
        y = conv(out, catw(wrb[(2 * blk) * 9:(2 * blk) * 9 + 9]))
        y = jnp.maximum(bn(y, 1 + 2 * blk), 0.0)
        y = conv(y, catw(wrb[(2 * blk + 1) * 9:(2 * blk + 1) * 9 + 9]))
        y = bn(y, 2 + 2 * blk)
        out = jnp.maximum(y + resid, 0.0)
    state_ref[...] = out

    pooled = jnp.dot(pm_ref[...], out.astype(bf), preferred_element_type=f32)

    w0b = w0_ref[...].astype(bf)
    d1 = w0b.shape[1]
    npos = pm_ref.shape[0] // n
    h1 = jnp.zeros((n, d1), f32)
    for s in range(npos):
        h1 = h1 + jnp.dot(pooled[s * n:(s + 1) * n, :].astype(bf),
                          w0b[s * c:(s + 1) * c, :],
                          preferred_element_type=f32)
    h1 = jnp.maximum(h1 + fb_ref[0:1, :d1], 0.0)
    h2 = jnp.dot(h1.astype(bf), w1_ref[...].astype(bf),
                 preferred_element_type=f32)
    h2 = jnp.maximum(h2 + fb_ref[1:2, :w1_ref.shape[1]], 0.0)
    o = jnp.dot(h2.astype(bf), w2_ref[...].astype(bf),
                preferred_element_type=f32)
    reward_ref[...] = o + fb_ref[2:3, :]


def _pool_const(n, h, w):
    ho, wo = h // 2, w // 2
    rp = np.arange(n * ho * wo)[:, None]
    q = np.arange(n * h * w)[None, :]
    n_r, s_r = rp % n, rp // n
    n_q, rem_q = q // (h * w), q % (h * w)
    hit = ((n_q == n_r) & ((rem_q // w) // 2 == s_r // wo)
           & ((rem_q % w) // 2 == s_r % wo))
    return np.where(hit, 0.25, 0.0).astype(jnp.bfloat16)


def kernel(x, stem_w, rb_w, bn_gamma, bn_beta, fc_w0, fc_w1, fc_w2, fc_b):
    n, c_in, h, w = x.shape
    c = stem_w.shape[2]
    nb = rb_w.shape[0] // 18
    rows = n * h * w
    prows = n * (h // 2) * (w // 2)
    npos = prows // n
    cp = ((c_in + 127) // 128) * 128

    x2d = x.transpose(0, 2, 3, 1).reshape(rows, c_in)
    x2d = jnp.pad(x2d, ((0, 0), (0, cp - c_in))).astype(jnp.bfloat16)
    ws = jnp.pad(stem_w, ((0, 0), (0, cp - c_in), (0, 0)))
    w0p = fc_w0.reshape(c, npos, fc_w0.shape[1]).transpose(1, 0, 2)
    w0p = w0p.reshape(npos * c, fc_w0.shape[1])
    pmat = _pool_const(n, h, w)

    out_pad = fc_w2.shape[1]
    state2d, reward_p = pl.pallas_call(
        functools.partial(_body, nb=nb, n=n, h=h, w=w, c=c, cp=cp),
        out_shape=(jax.ShapeDtypeStruct((rows, c), jnp.float32),
                   jax.ShapeDtypeStruct((n, out_pad), jnp.float32)),
        grid=(1,),
        in_specs=[
            pl.BlockSpec((rows, cp), lambda i: (0, 0)),
            pl.BlockSpec((9, cp, c), lambda i: (0, 0, 0)),
            pl.BlockSpec(rb_w.shape, lambda i: (0, 0, 0)),
            pl.BlockSpec(bn_gamma.shape, lambda i: (0, 0)),
            pl.BlockSpec(bn_beta.shape, lambda i: (0, 0)),
            pl.BlockSpec(pmat.shape, lambda i: (0, 0)),
            pl.BlockSpec(w0p.shape, lambda i: (0, 0)),
            pl.BlockSpec(fc_w1.shape, lambda i: (0, 0)),
            pl.BlockSpec(fc_w2.shape, lambda i: (0, 0)),
            pl.BlockSpec(fc_b.shape, lambda i: (0, 0)),
        ],
        out_specs=(pl.BlockSpec((rows, c), lambda i: (0, 0)),
                   pl.BlockSpec((n, out_pad), lambda i: (0, 0))),
        compiler_params=pltpu.CompilerParams(
            dimension_semantics=("arbitrary",)),
    )(x2d, ws, rb_w, bn_gamma, bn_beta, pmat, w0p, fc_w1, fc_w2, fc_b)

    state = state2d.reshape(n, h, w, c).transpose(0, 3, 1, 2)
    return state, reward_p[:, :_OUT_SIZE]
